# Initial kernel scaffold; baseline (speedup 1.0000x reference)
import jax
import jax.numpy as jnp
from jax.experimental import pallas as pl


def kernel(x, batch, seed_nodes, W_rel, W_root, b_rel):
    raise NotImplementedError("write your pallas kernel here")


# speedup vs baseline: 16.8807x; 16.8807x over previous
import functools

import jax
import jax.numpy as jnp
from jax import lax
from jax.experimental import pallas as pl
from jax.experimental.pallas import tpu as pltpu
from jax.experimental.pallas import tpu_sc as plsc

N = 100000
F = 128
NUM_SEG = 16
RATIO = 4

NC = 2
NS = 16
NW = NC * NS
L = 16

R = 800
NB = N // R
BLOCKS_PER_W = -(-NB // NW)


def _sc_segment_sum(x_flat, batch):
    mesh = plsc.VectorSubcoreMesh(core_axis_name="c", subcore_axis_name="s")

    @functools.partial(
        pl.kernel,
        mesh=mesh,
        out_type=jax.ShapeDtypeStruct((NW, NUM_SEG * F), jnp.float32),
        compiler_params=pltpu.CompilerParams(needs_layout_passes=False),
        scratch_types=[
            pltpu.VMEM((R * F,), jnp.float32),
            pltpu.VMEM((R,), jnp.int32),
            pltpu.VMEM((NUM_SEG * F,), jnp.float32),
        ],
    )
    def body(x_hbm, b_hbm, out_hbm, xb, bb, acc):
        wid = lax.axis_index("s") * NC + lax.axis_index("c")
        iota = lax.broadcasted_iota(jnp.int32, (L,), 0)
        zeros = jnp.zeros((L,), jnp.float32)

        def zero_body(i, _):
            acc[pl.ds(i * L, L)] = zeros
            return 0

        lax.fori_loop(0, NUM_SEG * F // L, zero_body, 0)

        def do_block(b):
            pltpu.sync_copy(x_hbm.at[pl.ds(b * (R * F), R * F)], xb)
            pltpu.sync_copy(b_hbm.at[pl.ds(b * R, R)], bb)

            def group_body(g, _):
                bvec = bb[pl.ds(g * L, L)]
                for j in range(L):
                    bj = bvec.at[jnp.full((L,), j, jnp.int32)].get(
                        mode="promise_in_bounds")
                    bidx = bj * F + iota
                    for c in range(F // L):
                        v = xb[pl.ds((g * L + j) * F + c * L, L)]
                        plsc.addupdate_scatter(acc, [bidx + c * L], v)
                return 0

            lax.fori_loop(0, R // L, group_body, 0)

        for k in range(BLOCKS_PER_W):
            b = wid + NW * k
            if (NW * k) + NW <= NB:
                do_block(b)
            else:
                @pl.when(b < NB)
                def _guarded():
                    do_block(b)

        pltpu.sync_copy(acc, out_hbm.at[wid])

    return body(x_flat, batch)


def _tc_combine(partials, seed_nodes, W_rel, W_root, b_rel2):
    def body(p_ref, seed_ref, wrel_ref, wroot_ref, brel_ref, out_ref):
        S = jnp.sum(p_ref[...], axis=0)
        A = lax.dot_general(S, wrel_ref[...], (((1,), (1,)), ((), ())),
                            preferred_element_type=jnp.float32)
        B = lax.dot_general(seed_ref[...], wroot_ref[...], (((1,), (1,)), ((), ())),
                            preferred_element_type=jnp.float32)
        out_ref[...] = (A[:, None, :] + B[None, :, :]
                        + brel_ref[...][None, :, :])

    return pl.pallas_call(
        body,
        out_shape=jax.ShapeDtypeStruct((NUM_SEG, RATIO, F), jnp.float32),
    )(partials, seed_nodes, W_rel, W_root, b_rel2)


def kernel(x, batch, seed_nodes, W_rel, W_root, b_rel):
    batch = batch.astype(jnp.int32)
    partials = _sc_segment_sum(x.reshape(-1), batch)
    partials = partials.reshape(NW, NUM_SEG, F)
    out3 = _tc_combine(partials, seed_nodes, W_rel, W_root,
                       b_rel.reshape(1, F))
    out = out3.reshape(NUM_SEG * RATIO, F)
    new_batchidx = jnp.repeat(jnp.arange(NUM_SEG, dtype=jnp.int32), RATIO)
    return out, new_batchidx


# speedup vs baseline: 30.3363x; 1.7971x over previous
import functools

import jax
import jax.numpy as jnp
from jax import lax
from jax.experimental import pallas as pl
from jax.experimental.pallas import tpu as pltpu
from jax.experimental.pallas import tpu_sc as plsc

N = 100000
F = 128
NUM_SEG = 16
RATIO = 4

NC = 2
NS = 16
NW = NC * NS
L = 16

R = 400
NB = N // R
BLOCKS_PER_W = -(-NB // NW)


def _sc_segment_sum(x, batch):
    mesh = plsc.VectorSubcoreMesh(core_axis_name="c", subcore_axis_name="s")

    @functools.partial(
        pl.kernel,
        mesh=mesh,
        out_type=jax.ShapeDtypeStruct((NW, NUM_SEG, F), jnp.float32),
        compiler_params=pltpu.CompilerParams(needs_layout_passes=False),
        scratch_types=[
            pltpu.VMEM((R, F), jnp.float32),
            pltpu.VMEM((R, F), jnp.float32),
            pltpu.VMEM((R,), jnp.int32),
            pltpu.VMEM((R,), jnp.int32),
            pltpu.VMEM((NUM_SEG, F), jnp.float32),
            pltpu.SemaphoreType.DMA,
            pltpu.SemaphoreType.DMA,
        ],
    )
    def body(x_hbm, b_hbm, out_hbm, xb0, xb1, bb0, bb1, acc, sem0, sem1):
        wid = lax.axis_index("s") * NC + lax.axis_index("c")
        iota = lax.broadcasted_iota(jnp.int32, (L,), 0)
        zeros = jnp.zeros((L,), jnp.float32)
        xbs, bbs, sems = (xb0, xb1), (bb0, bb1), (sem0, sem1)

        def zero_body(i, _):
            for c in range(F // L):
                acc[i, pl.ds(c * L, L)] = zeros
            return 0

        lax.fori_loop(0, NUM_SEG, zero_body, 0)

        def dma_start(b, s):
            pltpu.async_copy(x_hbm.at[pl.ds(b * R, R), :], xbs[s], sems[s])
            pltpu.async_copy(b_hbm.at[pl.ds(b * R, R)], bbs[s], sems[s])

        def dma_wait(b, s):
            pltpu.make_async_copy(
                x_hbm.at[pl.ds(b * R, R), :], xbs[s], sems[s]).wait()
            pltpu.make_async_copy(
                b_hbm.at[pl.ds(b * R, R)], bbs[s], sems[s]).wait()

        def compute(s):
            xb, bb = xbs[s], bbs[s]

            def group_body(g, _):
                row0 = g * L
                bvec = bb[pl.ds(row0, L)]
                b0 = bvec.at[jnp.zeros((L,), jnp.int32)].get(
                    mode="promise_in_bounds")
                uniform = jnp.all(bvec == b0)

                @pl.when(uniform)
                def _fast():
                    for c in range(F // L):
                        v = xb[row0, pl.ds(c * L, L)]
                        for j in range(1, L):
                            v = v + xb[row0 + j, pl.ds(c * L, L)]
                        plsc.addupdate_scatter(acc, [bvec, iota + c * L], v)

                @pl.when(jnp.logical_not(uniform))
                def _slow():
                    for j in range(L):
                        bj = bvec.at[jnp.full((L,), j, jnp.int32)].get(
                            mode="promise_in_bounds")
                        for c in range(F // L):
                            v = xb[row0 + j, pl.ds(c * L, L)]
                            plsc.addupdate_scatter(acc, [bj, iota + c * L], v)

                return 0

            lax.fori_loop(0, R // L, group_body, 0)

        dma_start(wid, 0)
        for k in range(BLOCKS_PER_W):
            b = wid + NW * k
            s = k % 2
            if (k + 1) < BLOCKS_PER_W:
                bn = wid + NW * (k + 1)
                if NW * (k + 1) + NW <= NB:
                    dma_start(bn, (k + 1) % 2)
                else:
                    @pl.when(bn < NB)
                    def _pref():
                        dma_start(bn, (k + 1) % 2)
            if NW * k + NW <= NB:
                dma_wait(b, s)
                compute(s)
            else:
                @pl.when(b < NB)
                def _tail():
                    dma_wait(b, s)
                    compute(s)

        pltpu.sync_copy(acc, out_hbm.at[wid])

    return body(x, batch)


def _tc_combine(partials, seed_nodes, W_rel, W_root, b_rel2):
    def body(p_ref, seed_ref, wrel_ref, wroot_ref, brel_ref, out_ref):
        S = jnp.sum(p_ref[...], axis=0)
        A = lax.dot_general(S, wrel_ref[...], (((1,), (1,)), ((), ())),
                            preferred_element_type=jnp.float32)
        B = lax.dot_general(seed_ref[...], wroot_ref[...], (((1,), (1,)), ((), ())),
                            preferred_element_type=jnp.float32)
        out_ref[...] = (A[:, None, :] + B[None, :, :]
                        + brel_ref[...][None, :, :])

    return pl.pallas_call(
        body,
        out_shape=jax.ShapeDtypeStruct((NUM_SEG, RATIO, F), jnp.float32),
    )(partials, seed_nodes, W_rel, W_root, b_rel2)


def kernel(x, batch, seed_nodes, W_rel, W_root, b_rel):
    batch = batch.astype(jnp.int32)
    partials = _sc_segment_sum(x, batch)
    out3 = _tc_combine(partials, seed_nodes, W_rel, W_root,
                       b_rel.reshape(1, F))
    out = out3.reshape(NUM_SEG * RATIO, F)
    new_batchidx = jnp.repeat(jnp.arange(NUM_SEG, dtype=jnp.int32), RATIO)
    return out, new_batchidx


# speedup vs baseline: 37.9414x; 1.2507x over previous
import functools

import jax
import jax.numpy as jnp
from jax import lax
from jax.experimental import pallas as pl
from jax.experimental.pallas import tpu as pltpu
from jax.experimental.pallas import tpu_sc as plsc

N = 100000
F = 128
NUM_SEG = 16
RATIO = 4

NC = 2
NS = 16
NW = NC * NS
L = 16

R = 400
NB = N // R
BLOCKS_PER_W = -(-NB // NW)


def _sc_segment_sum(x, batch):
    mesh = plsc.VectorSubcoreMesh(core_axis_name="c", subcore_axis_name="s")

    @functools.partial(
        pl.kernel,
        mesh=mesh,
        out_type=jax.ShapeDtypeStruct((NW, NUM_SEG, F), jnp.float32),
        compiler_params=pltpu.CompilerParams(needs_layout_passes=False),
        scratch_types=[
            pltpu.VMEM((R, F), jnp.float32),
            pltpu.VMEM((R, F), jnp.float32),
            pltpu.VMEM((R,), jnp.int32),
            pltpu.VMEM((R,), jnp.int32),
            pltpu.VMEM((NUM_SEG, F), jnp.float32),
            pltpu.SemaphoreType.DMA,
            pltpu.SemaphoreType.DMA,
        ],
    )
    def body(x_hbm, b_hbm, out_hbm, xb0, xb1, bb0, bb1, acc, sem0, sem1):
        wid = lax.axis_index("s") * NC + lax.axis_index("c")
        iota = lax.broadcasted_iota(jnp.int32, (L,), 0)
        zeros = jnp.zeros((L,), jnp.float32)
        xbs, bbs, sems = (xb0, xb1), (bb0, bb1), (sem0, sem1)

        def zero_body(i, _):
            for c in range(F // L):
                acc[i, pl.ds(c * L, L)] = zeros
            return 0

        lax.fori_loop(0, NUM_SEG, zero_body, 0)

        def dma_start(b, s):
            pltpu.async_copy(x_hbm.at[pl.ds(b * R, R), :], xbs[s], sems[s])
            pltpu.async_copy(b_hbm.at[pl.ds(b * R, R)], bbs[s], sems[s])

        def dma_wait(b, s):
            pltpu.make_async_copy(
                x_hbm.at[pl.ds(b * R, R), :], xbs[s], sems[s]).wait()
            pltpu.make_async_copy(
                b_hbm.at[pl.ds(b * R, R)], bbs[s], sems[s]).wait()

        def _tree_group_sum(xb, row0, c):
            v = [xb[row0 + j, pl.ds(c * L, L)] for j in range(L)]
            while len(v) > 1:
                v = [v[i] + v[i + 1] for i in range(0, len(v), 2)]
            return v[0]

        def compute(s):
            xb, bb = xbs[s], bbs[s]
            bhead = bb[pl.ds(0, L)]
            btail = bb[pl.ds(R - L, L)]
            block_uniform = jnp.all(bhead == btail)

            @pl.when(block_uniform)
            def _uniform_block():
                def gb(g, carry):
                    return tuple(
                        carry[c] + _tree_group_sum(xb, g * L, c)
                        for c in range(F // L))

                tot = lax.fori_loop(
                    0, R // L, gb,
                    tuple(jnp.zeros((L,), jnp.float32) for _ in range(F // L)))
                for c in range(F // L):
                    plsc.addupdate_scatter(acc, [bhead, iota + c * L], tot[c])

            @pl.when(jnp.logical_not(block_uniform))
            def _mixed_block():
                def group_body(g, _):
                    row0 = g * L
                    bvec = bb[pl.ds(row0, L)]
                    b0 = bvec.at[jnp.zeros((L,), jnp.int32)].get(
                        mode="promise_in_bounds")
                    uniform = jnp.all(bvec == b0)

                    @pl.when(uniform)
                    def _fast():
                        for c in range(F // L):
                            v = _tree_group_sum(xb, row0, c)
                            plsc.addupdate_scatter(acc, [bvec, iota + c * L], v)

                    @pl.when(jnp.logical_not(uniform))
                    def _slow():
                        for j in range(L):
                            bj = bvec.at[jnp.full((L,), j, jnp.int32)].get(
                                mode="promise_in_bounds")
                            for c in range(F // L):
                                v = xb[row0 + j, pl.ds(c * L, L)]
                                plsc.addupdate_scatter(
                                    acc, [bj, iota + c * L], v)

                    return 0

                lax.fori_loop(0, R // L, group_body, 0)

        dma_start(wid, 0)
        for k in range(BLOCKS_PER_W):
            b = wid + NW * k
            s = k % 2
            if (k + 1) < BLOCKS_PER_W:
                bn = wid + NW * (k + 1)
                if NW * (k + 1) + NW <= NB:
                    dma_start(bn, (k + 1) % 2)
                else:
                    @pl.when(bn < NB)
                    def _pref():
                        dma_start(bn, (k + 1) % 2)
            if NW * k + NW <= NB:
                dma_wait(b, s)
                compute(s)
            else:
                @pl.when(b < NB)
                def _tail():
                    dma_wait(b, s)
                    compute(s)

        pltpu.sync_copy(acc, out_hbm.at[wid])

    return body(x, batch)


def _tc_combine(partials, seed_nodes, W_rel, W_root, b_rel2):
    def body(p_ref, seed_ref, wrel_ref, wroot_ref, brel_ref, out_ref):
        S = jnp.sum(p_ref[...], axis=0)
        A = lax.dot_general(S, wrel_ref[...], (((1,), (1,)), ((), ())),
                            preferred_element_type=jnp.float32)
        B = lax.dot_general(seed_ref[...], wroot_ref[...], (((1,), (1,)), ((), ())),
                            preferred_element_type=jnp.float32)
        out_ref[...] = (A[:, None, :] + B[None, :, :]
                        + brel_ref[...][None, :, :])

    return pl.pallas_call(
        body,
        out_shape=jax.ShapeDtypeStruct((NUM_SEG, RATIO, F), jnp.float32),
    )(partials, seed_nodes, W_rel, W_root, b_rel2)


def kernel(x, batch, seed_nodes, W_rel, W_root, b_rel):
    batch = batch.astype(jnp.int32)
    partials = _sc_segment_sum(x, batch)
    out3 = _tc_combine(partials, seed_nodes, W_rel, W_root,
                       b_rel.reshape(1, F))
    out = out3.reshape(NUM_SEG * RATIO, F)
    new_batchidx = jnp.repeat(jnp.arange(NUM_SEG, dtype=jnp.int32), RATIO)
    return out, new_batchidx
